# baseline (speedup 1.0000x reference)
import functools

import jax
import jax.numpy as jnp
from jax import lax
from jax.experimental import pallas as pl
from jax.experimental.pallas import tpu as pltpu
from jax.experimental.pallas import tpu_sc as plsc

NN = 10000
EE = 320000
DD = 128
HH = 128
OO = 64

NC = 2
NS = 16
NW = NC * NS
CH = 128
EC = 2560
E_PAD = EC * CH
ECT = EC // NW
BLK = 40
CS = 40
CF = 120
N_TAB = 10240
NPT = N_TAB // NS
DUMMY = NN

BR = 2000
GRID = NN // BR

_mesh = plsc.VectorSubcoreMesh(core_axis_name="c", subcore_axis_name="s")


@functools.partial(
    pl.kernel,
    out_type=jax.ShapeDtypeStruct((NC, N_TAB), jnp.float32),
    mesh=_mesh,
    scratch_types=[
        pltpu.VMEM((ECT, CH), jnp.int32),
        pltpu.VMEM((CH,), jnp.float32),
        pltpu.VMEM((N_TAB // NS,), jnp.float32),
        pltpu.VMEM_SHARED((N_TAB,), jnp.float32),
    ],
)
def _sc_deg(col_hbm, out_hbm, cidx, ones_v, zbuf, dacc):
    c = lax.axis_index("c")
    s = lax.axis_index("s")
    w = c * NS + s
    zpt = N_TAB // NS

    def fill_zeros(i, carry):
        zbuf[pl.ds(i * 16, 16)] = jnp.zeros((16,), jnp.float32)
        return carry

    lax.fori_loop(0, zpt // 16, fill_zeros, 0)

    def fill_ones(i, carry):
        ones_v[pl.ds(i * 16, 16)] = jnp.ones((16,), jnp.float32)
        return carry

    lax.fori_loop(0, CH // 16, fill_ones, 0)

    pltpu.sync_copy(zbuf, dacc.at[pl.ds(s * zpt, zpt)])
    pltpu.sync_copy(col_hbm.at[pl.ds(w * ECT, ECT)], cidx)
    plsc.subcore_barrier()

    def body(j, carry):
        pltpu.sync_copy(ones_v, dacc.at[cidx.at[j]], add=True)
        return carry

    lax.fori_loop(0, ECT, body, 0)
    plsc.subcore_barrier()
    pltpu.sync_copy(dacc.at[pl.ds(s * zpt, zpt)], out_hbm.at[c, pl.ds(s * zpt, zpt)])


def _make_sc_agg(F):
    @functools.partial(
        pl.kernel,
        out_type=jax.ShapeDtypeStruct((NC, N_TAB, F), jnp.float32),
        mesh=_mesh,
        scratch_types=[
            pltpu.VMEM((BLK, CH), jnp.int32),
            pltpu.VMEM((BLK, CH), jnp.int32),
            pltpu.VMEM((CH, F), jnp.float32),
            pltpu.VMEM((CH, F), jnp.float32),
            pltpu.VMEM_SHARED((N_TAB, F), jnp.float32),
            pltpu.SemaphoreType.DMA,
            pltpu.SemaphoreType.DMA,
        ],
    )
    def agg(tab_hbm, row_hbm, col_hbm, zero_hbm, out_hbm,
            ridx, cidx, m0, m1, acc, sem0, sem1):
        c = lax.axis_index("c")
        s = lax.axis_index("s")

        pltpu.sync_copy(zero_hbm.at[pl.ds(s * NPT, NPT)], acc.at[pl.ds(s * NPT, NPT)])
        plsc.subcore_barrier()

        nblk = jnp.where(c == 0, CS // BLK, CF // BLK)
        tile_base = jnp.where(c == 0, s * CS, NS * CS + s * CF)

        def outer(b, carry):
            base = tile_base + b * BLK
            pltpu.sync_copy(row_hbm.at[pl.ds(base, BLK)], ridx)
            pltpu.sync_copy(col_hbm.at[pl.ds(base, BLK)], cidx)
            pltpu.async_copy(tab_hbm.at[ridx.at[0]], m0, sem0)

            def body(jj, inner_carry):
                j0 = 2 * jj
                j1 = j0 + 1
                pltpu.make_async_copy(tab_hbm.at[ridx.at[j0]], m0, sem0).wait()
                pltpu.async_copy(tab_hbm.at[ridx.at[j1]], m1, sem1)
                pltpu.sync_copy(m0, acc.at[cidx.at[j0]], add=True)
                pltpu.make_async_copy(tab_hbm.at[ridx.at[j1]], m1, sem1).wait()

                @pl.when(jj < BLK // 2 - 1)
                def _start_next():
                    pltpu.async_copy(tab_hbm.at[ridx.at[j0 + 2]], m0, sem0)

                pltpu.sync_copy(m1, acc.at[cidx.at[j1]], add=True)
                return inner_carry

            lax.fori_loop(0, BLK // 2, body, carry)
            return carry

        lax.fori_loop(0, nblk, outer, 0)
        plsc.subcore_barrier()
        pltpu.sync_copy(acc.at[pl.ds(s * NPT, NPT)], out_hbm.at[c, pl.ds(s * NPT, NPT)])

    return agg


_sc_agg128 = _make_sc_agg(HH)


def _dinv_of(deg_blk):
    return lax.rsqrt(deg_blk[0] + deg_blk[1] + 1.0)


def _mm1_body(deg_ref, x_ref, w_ref, o_ref):
    dinv = _dinv_of(deg_ref[...])
    h = jnp.dot(x_ref[...], w_ref[...], preferred_element_type=jnp.float32)
    o_ref[...] = dinv * h


_tc_mm1 = pl.pallas_call(
    _mm1_body,
    grid=(GRID,),
    in_specs=[
        pl.BlockSpec((NC, BR, 1), lambda i: (0, i, 0)),
        pl.BlockSpec((BR, DD), lambda i: (i, 0)),
        pl.BlockSpec((DD, HH), lambda i: (0, 0)),
    ],
    out_specs=pl.BlockSpec((BR, HH), lambda i: (i, 0)),
    out_shape=jax.ShapeDtypeStruct((NN, HH), jnp.float32),
)


def _mid_body(deg_ref, s1_ref, h1_ref, b1_ref, w2_ref, o1_ref, h2_ref):
    dinv = _dinv_of(deg_ref[...])
    s1 = s1_ref[...]
    agg = dinv * (s1[0] + s1[1] + h1_ref[...])
    o1 = jnp.maximum(agg + b1_ref[...], 0.0)
    o1_ref[...] = o1
    h2 = jnp.dot(o1, w2_ref[...], preferred_element_type=jnp.float32)
    h2_ref[...] = jnp.concatenate(
        [dinv * h2, jnp.zeros((BR, HH - OO), jnp.float32)], axis=1)


_tc_mid = pl.pallas_call(
    _mid_body,
    grid=(GRID,),
    in_specs=[
        pl.BlockSpec((NC, BR, 1), lambda i: (0, i, 0)),
        pl.BlockSpec((NC, BR, HH), lambda i: (0, i, 0)),
        pl.BlockSpec((BR, HH), lambda i: (i, 0)),
        pl.BlockSpec((1, HH), lambda i: (0, 0)),
        pl.BlockSpec((HH, OO), lambda i: (0, 0)),
    ],
    out_specs=[
        pl.BlockSpec((BR, HH), lambda i: (i, 0)),
        pl.BlockSpec((BR, HH), lambda i: (i, 0)),
    ],
    out_shape=[
        jax.ShapeDtypeStruct((NN, HH), jnp.float32),
        jax.ShapeDtypeStruct((NN, HH), jnp.float32),
    ],
)


def _final_body(deg_ref, s2_ref, h2_ref, b2_ref, o_ref):
    dinv = _dinv_of(deg_ref[...])
    s2 = s2_ref[...]
    logits = (dinv * (s2[0] + s2[1] + h2_ref[...]))[:, :OO] + b2_ref[...]
    m = jnp.max(logits, axis=1, keepdims=True)
    lse = jnp.log(jnp.sum(jnp.exp(logits - m), axis=1, keepdims=True)) + m
    o_ref[...] = logits - lse


_tc_final = pl.pallas_call(
    _final_body,
    grid=(GRID,),
    in_specs=[
        pl.BlockSpec((NC, BR, 1), lambda i: (0, i, 0)),
        pl.BlockSpec((NC, BR, HH), lambda i: (0, i, 0)),
        pl.BlockSpec((BR, HH), lambda i: (i, 0)),
        pl.BlockSpec((1, OO), lambda i: (0, 0)),
    ],
    out_specs=pl.BlockSpec((BR, OO), lambda i: (i, 0)),
    out_shape=jax.ShapeDtypeStruct((NN, OO), jnp.float32),
)


@jax.jit
def kernel(x, edge_index, W1, b1, W2, b2):
    row = edge_index[0]
    col = edge_index[1]
    pad = E_PAD - EE
    row2d = jnp.concatenate([row, jnp.zeros((pad,), row.dtype)]).reshape(EC, CH)
    col2d = jnp.concatenate([col, jnp.full((pad,), DUMMY, col.dtype)]).reshape(EC, CH)

    degp = _sc_deg(col2d)
    deg3 = degp.reshape(NC, N_TAB, 1)

    h1p = _tc_mm1(deg3, x, W1)
    z128 = jnp.zeros((N_TAB, HH), jnp.float32)
    s1 = _sc_agg128(h1p, row2d, col2d, z128)
    out1, h2p = _tc_mid(deg3, s1, h1p, b1.reshape(1, HH), W2)

    s2 = _sc_agg128(h2p, row2d, col2d, z128)
    out = _tc_final(deg3, s2, h2p, b2.reshape(1, OO))
    return (out, out1)


# speedup vs baseline: 1.1548x; 1.1548x over previous
import functools

import jax
import jax.numpy as jnp
from jax import lax
from jax.experimental import pallas as pl
from jax.experimental.pallas import tpu as pltpu
from jax.experimental.pallas import tpu_sc as plsc

NN = 10000
EE = 320000
DD = 128
HH = 128
OO = 64

NC = 2
NS = 16
NW = NC * NS
CH = 128
EC = 2560
E_PAD = EC * CH
ECT = EC // NW
BLK = 40
CS = 120
CF = 40
N_TAB = 10240
NPT = N_TAB // NS
DUMMY = NN

BR = 2000
GRID = NN // BR

_mesh = plsc.VectorSubcoreMesh(core_axis_name="c", subcore_axis_name="s")


@functools.partial(
    pl.kernel,
    out_type=jax.ShapeDtypeStruct((NC, N_TAB), jnp.float32),
    mesh=_mesh,
    scratch_types=[
        pltpu.VMEM((ECT, CH), jnp.int32),
        pltpu.VMEM((CH,), jnp.float32),
        pltpu.VMEM((N_TAB // NS,), jnp.float32),
        pltpu.VMEM_SHARED((N_TAB,), jnp.float32),
    ],
)
def _sc_deg(col_hbm, out_hbm, cidx, ones_v, zbuf, dacc):
    c = lax.axis_index("c")
    s = lax.axis_index("s")
    w = c * NS + s
    zpt = N_TAB // NS

    def fill_zeros(i, carry):
        zbuf[pl.ds(i * 16, 16)] = jnp.zeros((16,), jnp.float32)
        return carry

    lax.fori_loop(0, zpt // 16, fill_zeros, 0)

    def fill_ones(i, carry):
        ones_v[pl.ds(i * 16, 16)] = jnp.ones((16,), jnp.float32)
        return carry

    lax.fori_loop(0, CH // 16, fill_ones, 0)

    pltpu.sync_copy(zbuf, dacc.at[pl.ds(s * zpt, zpt)])
    pltpu.sync_copy(col_hbm.at[pl.ds(w * ECT, ECT)], cidx)
    plsc.subcore_barrier()

    def body(j, carry):
        pltpu.sync_copy(ones_v, dacc.at[cidx.at[j]], add=True)
        return carry

    lax.fori_loop(0, ECT, body, 0)
    plsc.subcore_barrier()
    pltpu.sync_copy(dacc.at[pl.ds(s * zpt, zpt)], out_hbm.at[c, pl.ds(s * zpt, zpt)])


def _make_sc_agg(F):
    @functools.partial(
        pl.kernel,
        out_type=jax.ShapeDtypeStruct((NC, N_TAB, F), jnp.float32),
        mesh=_mesh,
        scratch_types=[
            pltpu.VMEM((BLK, CH), jnp.int32),
            pltpu.VMEM((BLK, CH), jnp.int32),
            pltpu.VMEM((CH, F), jnp.float32),
            pltpu.VMEM((CH, F), jnp.float32),
            pltpu.VMEM_SHARED((N_TAB, F), jnp.float32),
            pltpu.SemaphoreType.DMA,
            pltpu.SemaphoreType.DMA,
        ],
    )
    def agg(tab_hbm, row_hbm, col_hbm, zero_hbm, out_hbm,
            ridx, cidx, m0, m1, acc, sem0, sem1):
        c = lax.axis_index("c")
        s = lax.axis_index("s")

        pltpu.sync_copy(zero_hbm.at[pl.ds(s * NPT, NPT)], acc.at[pl.ds(s * NPT, NPT)])
        plsc.subcore_barrier()

        nblk = jnp.where(c == 0, CS // BLK, CF // BLK)
        tile_base = jnp.where(c == 0, s * CS, NS * CS + s * CF)

        def outer(b, carry):
            base = tile_base + b * BLK
            pltpu.sync_copy(row_hbm.at[pl.ds(base, BLK)], ridx)
            pltpu.sync_copy(col_hbm.at[pl.ds(base, BLK)], cidx)
            pltpu.async_copy(tab_hbm.at[ridx.at[0]], m0, sem0)

            def body(jj, inner_carry):
                j0 = 2 * jj
                j1 = j0 + 1
                pltpu.make_async_copy(tab_hbm.at[ridx.at[j0]], m0, sem0).wait()
                pltpu.async_copy(tab_hbm.at[ridx.at[j1]], m1, sem1)
                pltpu.sync_copy(m0, acc.at[cidx.at[j0]], add=True)
                pltpu.make_async_copy(tab_hbm.at[ridx.at[j1]], m1, sem1).wait()

                @pl.when(jj < BLK // 2 - 1)
                def _start_next():
                    pltpu.async_copy(tab_hbm.at[ridx.at[j0 + 2]], m0, sem0)

                pltpu.sync_copy(m1, acc.at[cidx.at[j1]], add=True)
                return inner_carry

            lax.fori_loop(0, BLK // 2, body, carry)
            return carry

        lax.fori_loop(0, nblk, outer, 0)
        plsc.subcore_barrier()
        pltpu.sync_copy(acc.at[pl.ds(s * NPT, NPT)], out_hbm.at[c, pl.ds(s * NPT, NPT)])

    return agg


_sc_agg128 = _make_sc_agg(HH)


def _dinv_of(deg_blk):
    return lax.rsqrt(deg_blk[0] + deg_blk[1] + 1.0)


def _mm1_body(deg_ref, x_ref, w_ref, o_ref):
    dinv = _dinv_of(deg_ref[...])
    h = jnp.dot(x_ref[...], w_ref[...], preferred_element_type=jnp.float32)
    o_ref[...] = dinv * h


_tc_mm1 = pl.pallas_call(
    _mm1_body,
    grid=(GRID,),
    in_specs=[
        pl.BlockSpec((NC, BR, 1), lambda i: (0, i, 0)),
        pl.BlockSpec((BR, DD), lambda i: (i, 0)),
        pl.BlockSpec((DD, HH), lambda i: (0, 0)),
    ],
    out_specs=pl.BlockSpec((BR, HH), lambda i: (i, 0)),
    out_shape=jax.ShapeDtypeStruct((NN, HH), jnp.float32),
)


def _mid_body(deg_ref, s1_ref, h1_ref, b1_ref, w2_ref, o1_ref, h2_ref):
    dinv = _dinv_of(deg_ref[...])
    s1 = s1_ref[...]
    agg = dinv * (s1[0] + s1[1] + h1_ref[...])
    o1 = jnp.maximum(agg + b1_ref[...], 0.0)
    o1_ref[...] = o1
    h2 = jnp.dot(o1, w2_ref[...], preferred_element_type=jnp.float32)
    h2_ref[...] = jnp.concatenate(
        [dinv * h2, jnp.zeros((BR, HH - OO), jnp.float32)], axis=1)


_tc_mid = pl.pallas_call(
    _mid_body,
    grid=(GRID,),
    in_specs=[
        pl.BlockSpec((NC, BR, 1), lambda i: (0, i, 0)),
        pl.BlockSpec((NC, BR, HH), lambda i: (0, i, 0)),
        pl.BlockSpec((BR, HH), lambda i: (i, 0)),
        pl.BlockSpec((1, HH), lambda i: (0, 0)),
        pl.BlockSpec((HH, OO), lambda i: (0, 0)),
    ],
    out_specs=[
        pl.BlockSpec((BR, HH), lambda i: (i, 0)),
        pl.BlockSpec((BR, HH), lambda i: (i, 0)),
    ],
    out_shape=[
        jax.ShapeDtypeStruct((NN, HH), jnp.float32),
        jax.ShapeDtypeStruct((NN, HH), jnp.float32),
    ],
)


def _final_body(deg_ref, s2_ref, h2_ref, b2_ref, o_ref):
    dinv = _dinv_of(deg_ref[...])
    s2 = s2_ref[...]
    logits = (dinv * (s2[0] + s2[1] + h2_ref[...]))[:, :OO] + b2_ref[...]
    m = jnp.max(logits, axis=1, keepdims=True)
    lse = jnp.log(jnp.sum(jnp.exp(logits - m), axis=1, keepdims=True)) + m
    o_ref[...] = logits - lse


_tc_final = pl.pallas_call(
    _final_body,
    grid=(GRID,),
    in_specs=[
        pl.BlockSpec((NC, BR, 1), lambda i: (0, i, 0)),
        pl.BlockSpec((NC, BR, HH), lambda i: (0, i, 0)),
        pl.BlockSpec((BR, HH), lambda i: (i, 0)),
        pl.BlockSpec((1, OO), lambda i: (0, 0)),
    ],
    out_specs=pl.BlockSpec((BR, OO), lambda i: (i, 0)),
    out_shape=jax.ShapeDtypeStruct((NN, OO), jnp.float32),
)


@jax.jit
def kernel(x, edge_index, W1, b1, W2, b2):
    row = edge_index[0]
    col = edge_index[1]
    pad = E_PAD - EE
    row2d = jnp.concatenate([row, jnp.zeros((pad,), row.dtype)]).reshape(EC, CH)
    col2d = jnp.concatenate([col, jnp.full((pad,), DUMMY, col.dtype)]).reshape(EC, CH)

    degp = _sc_deg(col2d)
    deg3 = degp.reshape(NC, N_TAB, 1)

    h1p = _tc_mm1(deg3, x, W1)
    z128 = jnp.zeros((N_TAB, HH), jnp.float32)
    s1 = _sc_agg128(h1p, row2d, col2d, z128)
    out1, h2p = _tc_mid(deg3, s1, h1p, b1.reshape(1, HH), W2)

    s2 = _sc_agg128(h2p, row2d, col2d, z128)
    out = _tc_final(deg3, s2, h2p, b2.reshape(1, OO))
    return (out, out1)
